# baseline (speedup 1.0000x reference)
import functools

import jax
import jax.numpy as jnp
from jax import lax
from jax.experimental import pallas as pl
from jax.experimental.pallas import tpu as pltpu
from jax.experimental.pallas import tpu_sc as plsc

_B = 4096
_S = 200
_D = 64
_SR = _S // 8
_BC = _B // 128


@functools.cache
def _build_gather():
    info = plsc.get_sparse_core_info()
    nw = info.num_cores * info.num_subcores
    n_units = _SR * _BC
    u_per_w = n_units // nw
    assert u_per_w * nw == n_units
    mesh = plsc.VectorSubcoreMesh(core_axis_name="c", subcore_axis_name="s")

    @functools.partial(
        pl.kernel,
        mesh=mesh,
        out_type=jax.ShapeDtypeStruct((_S, _D // 8, _BC, 8, 128), jnp.float32),
        scratch_types=[
            pltpu.VMEM((8, 128), jnp.int32),
            pltpu.VMEM((128, _D), jnp.float32),
            pltpu.VMEM((128, _D), jnp.float32),
            pltpu.VMEM((_D // 8, 8, 128), jnp.float32),
            pltpu.VMEM((_D // 8, 8, 128), jnp.float32),
            pltpu.SemaphoreType.DMA,
            pltpu.SemaphoreType.DMA,
            pltpu.SemaphoreType.DMA,
            pltpu.SemaphoreType.DMA,
        ],
        compiler_params=pltpu.CompilerParams(
            needs_layout_passes=False, use_tc_tiling_on_sc=False
        ),
    )
    def gather_kernel(
        ids_hbm, table_hbm, out_hbm,
        idxb, g_a, g_b, t_a, t_b, sem_a, sem_b, sem_oa, sem_ob,
    ):
        wid = lax.axis_index("s") * info.num_cores + lax.axis_index("c")
        iota = lax.iota(jnp.int32, 16)

        def transpose_panel(g_buf, t_buf):
            for er in range(8):
                for rr in range(8):
                    cols = jnp.full((16,), er * 8 + rr, jnp.int32)
                    for b16 in range(8):
                        rows = iota + (b16 * 16)
                        v = plsc.load_gather(g_buf, [rows, cols])
                        t_buf[er, rr, pl.ds(b16 * 16, 16)] = v

        def step_panel(sl, sr, bc, u, g_buf, t_buf, sem_g, sem_o):
            s = sr * 8 + sl
            pltpu.make_async_copy(table_hbm.at[idxb.at[sl]], g_buf, sem_g).wait()

            @pl.when(u * 8 + sl >= 2)
            def _():
                pltpu.make_async_copy(t_buf, out_hbm.at[s, :, bc], sem_o).wait()

            transpose_panel(g_buf, t_buf)
            pltpu.async_copy(t_buf, out_hbm.at[s, :, bc], sem_o)

        def unit_body(u, carry):
            uid = wid * u_per_w + u
            sr = uid // _BC
            bc = uid % _BC
            pltpu.sync_copy(ids_hbm.at[sr, bc], idxb)
            pltpu.async_copy(table_hbm.at[idxb.at[0]], g_a, sem_a)

            def sl_body(sl, carry):
                @pl.when(sl < 7)
                def _():
                    @pl.when(lax.rem(sl, 2) == 0)
                    def _():
                        pltpu.async_copy(table_hbm.at[idxb.at[sl + 1]], g_b, sem_b)

                    @pl.when(lax.rem(sl, 2) == 1)
                    def _():
                        pltpu.async_copy(table_hbm.at[idxb.at[sl + 1]], g_a, sem_a)

                @pl.when(lax.rem(sl, 2) == 0)
                def _():
                    step_panel(sl, sr, bc, u, g_a, t_a, sem_a, sem_oa)

                @pl.when(lax.rem(sl, 2) == 1)
                def _():
                    step_panel(sl, sr, bc, u, g_b, t_b, sem_b, sem_ob)

                return carry

            lax.fori_loop(0, 8, sl_body, 0)
            return carry

        lax.fori_loop(0, u_per_w, unit_body, 0)
        last = wid * u_per_w + (u_per_w - 1)
        sr_l = last // _BC
        bc_l = last % _BC
        pltpu.make_async_copy(t_a, out_hbm.at[sr_l * 8 + 6, :, bc_l], sem_oa).wait()
        pltpu.make_async_copy(t_b, out_hbm.at[sr_l * 8 + 7, :, bc_l], sem_ob).wait()

    return gather_kernel


def kernel(input_ids, input_mask, emb_weight):
    ids4 = input_ids.T.reshape(_SR, 8, _BC, 128).transpose(0, 2, 1, 3)
    out5 = _build_gather()(ids4, emb_weight)
    out = out5.transpose(2, 4, 0, 1, 3).reshape(_B, _S, _D)
    return out, input_mask


# speedup vs baseline: 1.1893x; 1.1893x over previous
import functools

import jax
import jax.numpy as jnp
from jax import lax
from jax.experimental import pallas as pl
from jax.experimental.pallas import tpu as pltpu
from jax.experimental.pallas import tpu_sc as plsc

_B = 4096
_S = 200
_D = 64
_SR = _S // 8
_BC = _B // 128


@functools.cache
def _build_gather():
    info = plsc.get_sparse_core_info()
    nw = info.num_cores * info.num_subcores
    n_units = _SR * _BC
    u_per_w = n_units // nw
    assert u_per_w * nw == n_units
    mesh = plsc.VectorSubcoreMesh(core_axis_name="c", subcore_axis_name="s")

    @functools.partial(
        pl.kernel,
        mesh=mesh,
        out_type=jax.ShapeDtypeStruct((_S, _D // 8, _BC, 8, 128), jnp.float32),
        scratch_types=[
            pltpu.VMEM((8, 128), jnp.int32),
            pltpu.VMEM((128, _D), jnp.float32),
            pltpu.VMEM((128, _D), jnp.float32),
            pltpu.VMEM((_D // 8, 8, 128), jnp.float32),
            pltpu.VMEM((_D // 8, 8, 128), jnp.float32),
            pltpu.SemaphoreType.DMA,
            pltpu.SemaphoreType.DMA,
            pltpu.SemaphoreType.DMA,
            pltpu.SemaphoreType.DMA,
        ],
        compiler_params=pltpu.CompilerParams(
            needs_layout_passes=False, use_tc_tiling_on_sc=False
        ),
    )
    def gather_kernel(
        ids_hbm, table_hbm, out_hbm,
        idxb, g_a, g_b, t_a, t_b, sem_a, sem_b, sem_oa, sem_ob,
    ):
        wid = lax.axis_index("s") * info.num_cores + lax.axis_index("c")
        iota = lax.iota(jnp.int32, 16)

        def transpose_panel(g_buf, t_buf):
            for er in range(8):
                for rr0 in range(0, 8, 2):
                    vs = []
                    for rr in (rr0, rr0 + 1):
                        cols = jnp.full((16,), er * 8 + rr, jnp.int32)
                        for b16 in range(8):
                            rows = iota + (b16 * 16)
                            vs.append(plsc.load_gather(g_buf, [rows, cols]))
                    k = 0
                    for rr in (rr0, rr0 + 1):
                        for b16 in range(8):
                            t_buf[er, rr, pl.ds(b16 * 16, 16)] = vs[k]
                            k += 1

        def step_panel(sl, sr, bc, u, g_buf, t_buf, sem_g, sem_o):
            s = sr * 8 + sl
            pltpu.make_async_copy(table_hbm.at[idxb.at[sl]], g_buf, sem_g).wait()

            @pl.when(u * 8 + sl >= 2)
            def _():
                pltpu.make_async_copy(t_buf, out_hbm.at[s, :, bc], sem_o).wait()

            transpose_panel(g_buf, t_buf)
            pltpu.async_copy(t_buf, out_hbm.at[s, :, bc], sem_o)

        def unit_body(u, carry):
            uid = wid * u_per_w + u
            sr = uid // _BC
            bc = uid % _BC
            pltpu.sync_copy(ids_hbm.at[sr, bc], idxb)
            pltpu.async_copy(table_hbm.at[idxb.at[0]], g_a, sem_a)

            def sl_body(sl, carry):
                @pl.when(sl < 7)
                def _():
                    @pl.when(lax.rem(sl, 2) == 0)
                    def _():
                        pltpu.async_copy(table_hbm.at[idxb.at[sl + 1]], g_b, sem_b)

                    @pl.when(lax.rem(sl, 2) == 1)
                    def _():
                        pltpu.async_copy(table_hbm.at[idxb.at[sl + 1]], g_a, sem_a)

                @pl.when(lax.rem(sl, 2) == 0)
                def _():
                    step_panel(sl, sr, bc, u, g_a, t_a, sem_a, sem_oa)

                @pl.when(lax.rem(sl, 2) == 1)
                def _():
                    step_panel(sl, sr, bc, u, g_b, t_b, sem_b, sem_ob)

                return carry

            lax.fori_loop(0, 8, sl_body, 0)
            return carry

        lax.fori_loop(0, u_per_w, unit_body, 0)
        last = wid * u_per_w + (u_per_w - 1)
        sr_l = last // _BC
        bc_l = last % _BC
        pltpu.make_async_copy(t_a, out_hbm.at[sr_l * 8 + 6, :, bc_l], sem_oa).wait()
        pltpu.make_async_copy(t_b, out_hbm.at[sr_l * 8 + 7, :, bc_l], sem_ob).wait()

    return gather_kernel


def kernel(input_ids, input_mask, emb_weight):
    ids4 = input_ids.T.reshape(_SR, 8, _BC, 128).transpose(0, 2, 1, 3)
    out5 = _build_gather()(ids4, emb_weight)
    out = out5.transpose(2, 4, 0, 1, 3).reshape(_B, _S, _D)
    return out, input_mask


# speedup vs baseline: 1.2109x; 1.0182x over previous
import functools

import jax
import jax.numpy as jnp
from jax import lax
from jax.experimental import pallas as pl
from jax.experimental.pallas import tpu as pltpu
from jax.experimental.pallas import tpu_sc as plsc

_B = 4096
_S = 200
_D = 64
_SR = _S // 8
_BC = _B // 128


@functools.cache
def _build_gather():
    info = plsc.get_sparse_core_info()
    nw = info.num_cores * info.num_subcores
    n_units = _SR * _BC
    u_per_w = n_units // nw
    assert u_per_w * nw == n_units
    mesh = plsc.VectorSubcoreMesh(core_axis_name="c", subcore_axis_name="s")

    @functools.partial(
        pl.kernel,
        mesh=mesh,
        out_type=jax.ShapeDtypeStruct((_S, _D // 8, _BC, 8, 128), jnp.float32),
        scratch_types=[
            pltpu.VMEM((8, 128), jnp.int32),
            pltpu.VMEM((128, _D), jnp.float32),
            pltpu.VMEM((128, _D), jnp.float32),
            pltpu.VMEM((_D // 8, 8, 128), jnp.float32),
            pltpu.VMEM((_D // 8, 8, 128), jnp.float32),
            pltpu.SemaphoreType.DMA,
            pltpu.SemaphoreType.DMA,
            pltpu.SemaphoreType.DMA,
            pltpu.SemaphoreType.DMA,
        ],
        compiler_params=pltpu.CompilerParams(
            needs_layout_passes=False, use_tc_tiling_on_sc=False
        ),
    )
    def gather_kernel(
        ids_hbm, table_hbm, out_hbm,
        idxb, g_a, g_b, t_a, t_b, sem_a, sem_b, sem_oa, sem_ob,
    ):
        wid = lax.axis_index("s") * info.num_cores + lax.axis_index("c")
        iota = lax.iota(jnp.int32, 16)

        def transpose_panel(g_buf, t_buf):
            def gathers(e):
                cols = jnp.full((16,), e, jnp.int32)
                return [
                    plsc.load_gather(g_buf, [iota + (b16 * 16), cols])
                    for b16 in range(8)
                ]

            prev = gathers(0)
            for e in range(1, 64):
                er, rr = (e - 1) // 8, (e - 1) % 8
                cols = jnp.full((16,), e, jnp.int32)
                cur = []
                for b16 in range(8):
                    cur.append(plsc.load_gather(g_buf, [iota + (b16 * 16), cols]))
                    t_buf[er, rr, pl.ds(b16 * 16, 16)] = prev[b16]
                prev = cur
            for b16 in range(8):
                t_buf[7, 7, pl.ds(b16 * 16, 16)] = prev[b16]

        def step_panel(sl, sr, bc, u, g_buf, t_buf, sem_g, sem_o):
            s = sr * 8 + sl
            pltpu.make_async_copy(table_hbm.at[idxb.at[sl]], g_buf, sem_g).wait()

            @pl.when(u * 8 + sl >= 2)
            def _():
                pltpu.make_async_copy(t_buf, out_hbm.at[s, :, bc], sem_o).wait()

            transpose_panel(g_buf, t_buf)
            pltpu.async_copy(t_buf, out_hbm.at[s, :, bc], sem_o)

        def unit_body(u, carry):
            uid = wid * u_per_w + u
            sr = uid // _BC
            bc = uid % _BC
            pltpu.sync_copy(ids_hbm.at[sr, bc], idxb)
            pltpu.async_copy(table_hbm.at[idxb.at[0]], g_a, sem_a)

            def sl_body(sl, carry):
                @pl.when(sl < 7)
                def _():
                    @pl.when(lax.rem(sl, 2) == 0)
                    def _():
                        pltpu.async_copy(table_hbm.at[idxb.at[sl + 1]], g_b, sem_b)

                    @pl.when(lax.rem(sl, 2) == 1)
                    def _():
                        pltpu.async_copy(table_hbm.at[idxb.at[sl + 1]], g_a, sem_a)

                @pl.when(lax.rem(sl, 2) == 0)
                def _():
                    step_panel(sl, sr, bc, u, g_a, t_a, sem_a, sem_oa)

                @pl.when(lax.rem(sl, 2) == 1)
                def _():
                    step_panel(sl, sr, bc, u, g_b, t_b, sem_b, sem_ob)

                return carry

            lax.fori_loop(0, 8, sl_body, 0)
            return carry

        lax.fori_loop(0, u_per_w, unit_body, 0)
        last = wid * u_per_w + (u_per_w - 1)
        sr_l = last // _BC
        bc_l = last % _BC
        pltpu.make_async_copy(t_a, out_hbm.at[sr_l * 8 + 6, :, bc_l], sem_oa).wait()
        pltpu.make_async_copy(t_b, out_hbm.at[sr_l * 8 + 7, :, bc_l], sem_ob).wait()

    return gather_kernel


def kernel(input_ids, input_mask, emb_weight):
    ids4 = input_ids.T.reshape(_SR, 8, _BC, 128).transpose(0, 2, 1, 3)
    out5 = _build_gather()(ids4, emb_weight)
    out = out5.transpose(2, 4, 0, 1, 3).reshape(_B, _S, _D)
    return out, input_mask


# speedup vs baseline: 2.0214x; 1.6693x over previous
import functools

import jax
import jax.numpy as jnp
from jax import lax
from jax.experimental import pallas as pl
from jax.experimental.pallas import tpu as pltpu
from jax.experimental.pallas import tpu_sc as plsc

_B = 4096
_S = 200
_D = 64
_SR = _S // 8
_BC = _B // 128


@functools.cache
def _build_gather():
    info = plsc.get_sparse_core_info()
    nw = info.num_cores * info.num_subcores
    n_units = _SR * _BC
    u_per_w = n_units // nw
    assert u_per_w * nw == n_units
    mesh = plsc.VectorSubcoreMesh(core_axis_name="c", subcore_axis_name="s")

    @functools.partial(
        pl.kernel,
        mesh=mesh,
        out_type=jax.ShapeDtypeStruct((_S, _D // 8, _BC, 8, 128), jnp.float32),
        scratch_types=[
            pltpu.VMEM((8, 128), jnp.int32),
            pltpu.VMEM((128, _D), jnp.float32),
            pltpu.VMEM((128, _D), jnp.float32),
            pltpu.VMEM((_D // 8, 8, 128), jnp.float32),
            pltpu.VMEM((_D // 8, 8, 128), jnp.float32),
            pltpu.SemaphoreType.DMA,
            pltpu.SemaphoreType.DMA,
            pltpu.SemaphoreType.DMA,
            pltpu.SemaphoreType.DMA,
        ],
        compiler_params=pltpu.CompilerParams(
            needs_layout_passes=False, use_tc_tiling_on_sc=False
        ),
    )
    def gather_kernel(
        ids_hbm, table_hbm, out_hbm,
        idxb, g_a, g_b, t_a, t_b, sem_a, sem_b, sem_oa, sem_ob,
    ):
        wid = lax.axis_index("s") * info.num_cores + lax.axis_index("c")
        iota = lax.iota(jnp.int32, 16)

        def transpose_panel(g_buf, t_buf):
            rows_l = [iota + (b16 * 16) for b16 in range(8)]

            def idxs(d):
                f = jnp.bitwise_and(d + iota, 63)
                return f, jnp.right_shift(f, 3), jnp.bitwise_and(f, 7)

            f0, er0, rr0 = idxs(0)
            prev = [
                (plsc.load_gather(g_buf, [rows_l[b], f0]), er0, rr0, rows_l[b])
                for b in range(8)
            ]
            for d in range(1, 64):
                f, er_i, rr_i = idxs(d)
                cur = []
                for b in range(8):
                    v = plsc.load_gather(g_buf, [rows_l[b], f])
                    pv, per, prr, prow = prev[b]
                    plsc.store_scatter(t_buf, [per, prr, prow], pv)
                    cur.append((v, er_i, rr_i, rows_l[b]))
                prev = cur
            for b in range(8):
                pv, per, prr, prow = prev[b]
                plsc.store_scatter(t_buf, [per, prr, prow], pv)

        def step_panel(sl, sr, bc, u, g_buf, t_buf, sem_g, sem_o):
            s = sr * 8 + sl
            pltpu.make_async_copy(table_hbm.at[idxb.at[sl]], g_buf, sem_g).wait()

            @pl.when(u * 8 + sl >= 2)
            def _():
                pltpu.make_async_copy(t_buf, out_hbm.at[s, :, bc], sem_o).wait()

            transpose_panel(g_buf, t_buf)
            pltpu.async_copy(t_buf, out_hbm.at[s, :, bc], sem_o)

        def unit_body(u, carry):
            uid = wid * u_per_w + u
            sr = uid // _BC
            bc = uid % _BC
            pltpu.sync_copy(ids_hbm.at[sr, bc], idxb)
            pltpu.async_copy(table_hbm.at[idxb.at[0]], g_a, sem_a)

            def sl_body(sl, carry):
                @pl.when(sl < 7)
                def _():
                    @pl.when(lax.rem(sl, 2) == 0)
                    def _():
                        pltpu.async_copy(table_hbm.at[idxb.at[sl + 1]], g_b, sem_b)

                    @pl.when(lax.rem(sl, 2) == 1)
                    def _():
                        pltpu.async_copy(table_hbm.at[idxb.at[sl + 1]], g_a, sem_a)

                @pl.when(lax.rem(sl, 2) == 0)
                def _():
                    step_panel(sl, sr, bc, u, g_a, t_a, sem_a, sem_oa)

                @pl.when(lax.rem(sl, 2) == 1)
                def _():
                    step_panel(sl, sr, bc, u, g_b, t_b, sem_b, sem_ob)

                return carry

            lax.fori_loop(0, 8, sl_body, 0)
            return carry

        lax.fori_loop(0, u_per_w, unit_body, 0)
        last = wid * u_per_w + (u_per_w - 1)
        sr_l = last // _BC
        bc_l = last % _BC
        pltpu.make_async_copy(t_a, out_hbm.at[sr_l * 8 + 6, :, bc_l], sem_oa).wait()
        pltpu.make_async_copy(t_b, out_hbm.at[sr_l * 8 + 7, :, bc_l], sem_ob).wait()

    return gather_kernel


def kernel(input_ids, input_mask, emb_weight):
    ids4 = input_ids.T.reshape(_SR, 8, _BC, 128).transpose(0, 2, 1, 3)
    out5 = _build_gather()(ids4, emb_weight)
    out = out5.transpose(2, 4, 0, 1, 3).reshape(_B, _S, _D)
    return out, input_mask


# speedup vs baseline: 2.0472x; 1.0128x over previous
import functools

import jax
import jax.numpy as jnp
from jax import lax
from jax.experimental import pallas as pl
from jax.experimental.pallas import tpu as pltpu
from jax.experimental.pallas import tpu_sc as plsc

_B = 4096
_S = 200
_D = 64
_SR = _S // 8
_BC = _B // 128


@functools.cache
def _build_gather():
    info = plsc.get_sparse_core_info()
    nw = info.num_cores * info.num_subcores
    n_units = _SR * _BC
    u_per_w = n_units // nw
    assert u_per_w * nw == n_units
    mesh = plsc.VectorSubcoreMesh(core_axis_name="c", subcore_axis_name="s")

    @functools.partial(
        pl.kernel,
        mesh=mesh,
        out_type=jax.ShapeDtypeStruct((_S, _D // 8, _BC, 8, 128), jnp.float32),
        scratch_types=[
            pltpu.VMEM((u_per_w, 8, 128), jnp.int32),
            pltpu.VMEM((128, _D), jnp.float32),
            pltpu.VMEM((128, _D), jnp.float32),
            pltpu.VMEM((_D // 8, 8, 128), jnp.float32),
            pltpu.VMEM((_D // 8, 8, 128), jnp.float32),
            pltpu.SemaphoreType.DMA,
            pltpu.SemaphoreType.DMA,
            pltpu.SemaphoreType.DMA,
            pltpu.SemaphoreType.DMA,
        ],
        compiler_params=pltpu.CompilerParams(
            needs_layout_passes=False, use_tc_tiling_on_sc=False
        ),
    )
    def gather_kernel(
        ids_hbm, table_hbm, out_hbm,
        idxb, g_a, g_b, t_a, t_b, sem_a, sem_b, sem_oa, sem_ob,
    ):
        wid = lax.axis_index("s") * info.num_cores + lax.axis_index("c")
        iota = lax.iota(jnp.int32, 16)
        pltpu.sync_copy(ids_hbm.at[pl.ds(wid * u_per_w, u_per_w)], idxb)

        def transpose_panel(g_buf, t_buf):
            rows_l = [iota + (b16 * 16) for b16 in range(8)]

            def idxs(d):
                f = jnp.bitwise_and(d + iota, 63)
                return f, jnp.right_shift(f, 3), jnp.bitwise_and(f, 7)

            f0, er0, rr0 = idxs(0)
            prev = [
                (plsc.load_gather(g_buf, [rows_l[b], f0]), er0, rr0, rows_l[b])
                for b in range(8)
            ]
            for d in range(1, 64):
                f, er_i, rr_i = idxs(d)
                cur = []
                for b in range(8):
                    v = plsc.load_gather(g_buf, [rows_l[b], f])
                    pv, per, prr, prow = prev[b]
                    plsc.store_scatter(t_buf, [per, prr, prow], pv)
                    cur.append((v, er_i, rr_i, rows_l[b]))
                prev = cur
            for b in range(8):
                pv, per, prr, prow = prev[b]
                plsc.store_scatter(t_buf, [per, prr, prow], pv)

        def step_panel(sl, sr, bc, u, g_buf, t_buf, sem_g, sem_o):
            s = sr * 8 + sl
            pltpu.make_async_copy(table_hbm.at[idxb.at[u, sl]], g_buf, sem_g).wait()

            @pl.when(u * 8 + sl >= 2)
            def _():
                pltpu.make_async_copy(t_buf, out_hbm.at[s, :, bc], sem_o).wait()

            transpose_panel(g_buf, t_buf)
            pltpu.async_copy(t_buf, out_hbm.at[s, :, bc], sem_o)

        def unit_body(u, carry):
            uid = wid * u_per_w + u
            sr = uid // _BC
            bc = uid % _BC
            pltpu.async_copy(table_hbm.at[idxb.at[u, 0]], g_a, sem_a)

            def sl_body(sl, carry):
                @pl.when(sl < 7)
                def _():
                    @pl.when(lax.rem(sl, 2) == 0)
                    def _():
                        pltpu.async_copy(table_hbm.at[idxb.at[u, sl + 1]], g_b, sem_b)

                    @pl.when(lax.rem(sl, 2) == 1)
                    def _():
                        pltpu.async_copy(table_hbm.at[idxb.at[u, sl + 1]], g_a, sem_a)

                @pl.when(lax.rem(sl, 2) == 0)
                def _():
                    step_panel(sl, sr, bc, u, g_a, t_a, sem_a, sem_oa)

                @pl.when(lax.rem(sl, 2) == 1)
                def _():
                    step_panel(sl, sr, bc, u, g_b, t_b, sem_b, sem_ob)

                return carry

            lax.fori_loop(0, 8, sl_body, 0)
            return carry

        lax.fori_loop(0, u_per_w, unit_body, 0)
        last = wid * u_per_w + (u_per_w - 1)
        sr_l = last // _BC
        bc_l = last % _BC
        pltpu.make_async_copy(t_a, out_hbm.at[sr_l * 8 + 6, :, bc_l], sem_oa).wait()
        pltpu.make_async_copy(t_b, out_hbm.at[sr_l * 8 + 7, :, bc_l], sem_ob).wait()

    return gather_kernel


def kernel(input_ids, input_mask, emb_weight):
    ids3 = (
        input_ids.T.reshape(_SR, 8, _BC, 128)
        .transpose(0, 2, 1, 3)
        .reshape(_SR * _BC, 8, 128)
    )
    out5 = _build_gather()(ids3, emb_weight)
    out = out5.transpose(2, 4, 0, 1, 3).reshape(_B, _S, _D)
    return out, input_mask


# speedup vs baseline: 2.1238x; 1.0374x over previous
import functools

import jax
import jax.numpy as jnp
from jax import lax
from jax.experimental import pallas as pl
from jax.experimental.pallas import tpu as pltpu
from jax.experimental.pallas import tpu_sc as plsc

_B = 4096
_S = 200
_D = 64
_SR = _S // 8
_BC = _B // 128


@functools.cache
def _build_gather():
    info = plsc.get_sparse_core_info()
    nw = info.num_cores * info.num_subcores
    n_units = _SR * _BC
    u_per_w = n_units // nw
    assert u_per_w * nw == n_units
    mesh = plsc.VectorSubcoreMesh(core_axis_name="c", subcore_axis_name="s")

    @functools.partial(
        pl.kernel,
        mesh=mesh,
        out_type=jax.ShapeDtypeStruct((_S, _D // 8, _BC, 8, 128), jnp.float32),
        scratch_types=[
            pltpu.VMEM((u_per_w, 8, 128), jnp.int32),
            pltpu.VMEM((128, _D), jnp.float32),
            pltpu.VMEM((128, _D), jnp.float32),
            pltpu.VMEM((_D // 8, 8, 128), jnp.float32),
            pltpu.VMEM((_D // 8, 8, 128), jnp.float32),
            pltpu.SemaphoreType.DMA,
            pltpu.SemaphoreType.DMA,
            pltpu.SemaphoreType.DMA,
            pltpu.SemaphoreType.DMA,
        ],
        compiler_params=pltpu.CompilerParams(
            needs_layout_passes=False, use_tc_tiling_on_sc=False
        ),
    )
    def gather_kernel(
        ids_hbm, table_hbm, out_hbm,
        idxb, g_a, g_b, t_a, t_b, sem_a, sem_b, sem_oa, sem_ob,
    ):
        wid = lax.axis_index("s") * info.num_cores + lax.axis_index("c")
        iota = lax.iota(jnp.int32, 16)
        pltpu.sync_copy(ids_hbm.at[pl.ds(wid * u_per_w, u_per_w)], idxb)

        def transpose_panel(g_buf, t_buf):
            rows_l = [iota + (b16 * 16) for b16 in range(8)]

            def idxs(d):
                f = jnp.bitwise_and(d + iota, 63)
                return f, jnp.right_shift(f, 3), jnp.bitwise_and(f, 7)

            f0, er0, rr0 = idxs(0)
            prev = [
                (plsc.load_gather(g_buf, [rows_l[b], f0]), er0, rr0, rows_l[b])
                for b in range(8)
            ]
            for d in range(1, 64):
                f, er_i, rr_i = idxs(d)
                cur = []
                for b in range(8):
                    v = plsc.load_gather(g_buf, [rows_l[b], f])
                    pv, per, prr, prow = prev[b]
                    plsc.store_scatter(t_buf, [per, prr, prow], pv)
                    cur.append((v, er_i, rr_i, rows_l[b]))
                prev = cur
            for b in range(8):
                pv, per, prr, prow = prev[b]
                plsc.store_scatter(t_buf, [per, prr, prow], pv)

        def step_panel(sl, sr, bc, u, g_buf, t_buf, sem_g, sem_o):
            s = sr * 8 + sl
            pltpu.make_async_copy(table_hbm.at[idxb.at[u, sl]], g_buf, sem_g).wait()

            @pl.when(u * 8 + sl >= 2)
            def _():
                pltpu.make_async_copy(t_buf, out_hbm.at[s, :, bc], sem_o).wait()

            transpose_panel(g_buf, t_buf)
            pltpu.async_copy(t_buf, out_hbm.at[s, :, bc], sem_o)

        def unit_body(u, carry):
            uid = wid * u_per_w + u
            sr = uid // _BC
            bc = uid % _BC
            @pl.when(u == 0)
            def _():
                pltpu.async_copy(table_hbm.at[idxb.at[0, 0]], g_a, sem_a)

            def sl_body(sl, carry):
                @pl.when(sl < 7)
                def _():
                    @pl.when(lax.rem(sl, 2) == 0)
                    def _():
                        pltpu.async_copy(table_hbm.at[idxb.at[u, sl + 1]], g_b, sem_b)

                    @pl.when(lax.rem(sl, 2) == 1)
                    def _():
                        pltpu.async_copy(table_hbm.at[idxb.at[u, sl + 1]], g_a, sem_a)

                @pl.when(jnp.logical_and(sl == 7, u < u_per_w - 1))
                def _():
                    pltpu.async_copy(table_hbm.at[idxb.at[u + 1, 0]], g_a, sem_a)

                @pl.when(lax.rem(sl, 2) == 0)
                def _():
                    step_panel(sl, sr, bc, u, g_a, t_a, sem_a, sem_oa)

                @pl.when(lax.rem(sl, 2) == 1)
                def _():
                    step_panel(sl, sr, bc, u, g_b, t_b, sem_b, sem_ob)

                return carry

            lax.fori_loop(0, 8, sl_body, 0)
            return carry

        lax.fori_loop(0, u_per_w, unit_body, 0)
        last = wid * u_per_w + (u_per_w - 1)
        sr_l = last // _BC
        bc_l = last % _BC
        pltpu.make_async_copy(t_a, out_hbm.at[sr_l * 8 + 6, :, bc_l], sem_oa).wait()
        pltpu.make_async_copy(t_b, out_hbm.at[sr_l * 8 + 7, :, bc_l], sem_ob).wait()

    return gather_kernel


def kernel(input_ids, input_mask, emb_weight):
    ids3 = (
        input_ids.T.reshape(_SR, 8, _BC, 128)
        .transpose(0, 2, 1, 3)
        .reshape(_SR * _BC, 8, 128)
    )
    out5 = _build_gather()(ids3, emb_weight)
    out = out5.transpose(2, 4, 0, 1, 3).reshape(_B, _S, _D)
    return out, input_mask
